# baseline (speedup 1.0000x reference)
import math

import jax
import jax.numpy as jnp
from jax.experimental import pallas as pl
from jax.experimental.pallas import tpu as pltpu

B, Q, P, NE = 4, 256, 16384, 64
NO_ELECTRON_WEIGHT = 0.1
HUBER_DELTA = 0.1

TP = 2048
NPT = P // TP

_HIGH = jax.lax.Precision.HIGHEST


def _bce(x, z):
    return jnp.maximum(x, 0.0) - x * z + jnp.log1p(jnp.exp(-jnp.abs(x)))


def _loss_kernel(mi_ref, logits_ref, pos_ref, chol_ref, tpos_ref,
                 seg_ref, true_ref, acc_ref, total_ref):
    b = pl.program_id(0)
    pt = pl.program_id(1)

    @pl.when(jnp.logical_and(b == 0, pt == 0))
    def _init():
        acc_ref[...] = jnp.zeros_like(acc_ref)

    pi = mi_ref[0, 0:1, :].astype(jnp.int32)
    ti = mi_ref[0, 1:2, :].astype(jnp.int32)
    iota_q = jax.lax.broadcasted_iota(jnp.int32, (Q, NE), 0)
    sel_p = (iota_q == pi).astype(jnp.float32)
    iota_e = jax.lax.broadcasted_iota(jnp.int32, (NE, NE), 0)
    sel_t = (iota_e == ti).astype(jnp.float32)

    seg = seg_ref[0]
    tru = true_ref[0]

    x = jax.lax.dot_general(seg, sel_p, (((1,), (0,)), ((), ())),
                            precision=_HIGH)
    t = jax.lax.dot_general(tru, sel_t, (((1,), (0,)), ((), ())),
                            precision=_HIGH)

    bce_rows = jnp.sum(_bce(x, t), axis=0, keepdims=True)
    acc_ref[b, 0:1, 0:NE] += bce_rows

    m = jnp.max(x, axis=1, keepdims=True)
    ex = jnp.exp(x - m)
    s = jnp.sum(ex, axis=1, keepdims=True)
    sm = ex / s
    acc_ref[b, 1:2, 0:NE] += jnp.sum(2.0 * sm * t, axis=0, keepdims=True)
    acc_ref[b, 2:3, 0:NE] += jnp.sum(sm + t, axis=0, keepdims=True)

    @pl.when(pt == 0)
    def _small_losses():
        xq = logits_ref[0]
        label = jnp.sum(sel_p, axis=1, keepdims=True)
        w = jnp.where(label > 0, 1.0, NO_ELECTRON_WEIGHT)
        acc_ref[b, 3:4, 0:1] += jnp.sum(w * _bce(xq, label), axis=0,
                                        keepdims=True)

        pos = pos_ref[0]
        chol = chol_ref[0]
        tpos = tpos_ref[0]
        pp = jax.lax.dot_general(sel_p, pos, (((0,), (0,)), ((), ())),
                                 precision=_HIGH)
        lg = jax.lax.dot_general(sel_p, chol, (((0,), (0,)), ((), ())),
                                 precision=_HIGH)
        tp = jax.lax.dot_general(sel_t, tpos, (((0,), (0,)), ((), ())),
                                 precision=_HIGH)

        d = tp - pp
        l00 = lg[:, 0:1]
        l10 = lg[:, 2:3]
        l11 = lg[:, 3:4]
        z0 = d[:, 0:1] / l00
        z1 = (d[:, 1:2] - l10 * z0) / l11
        maha = z0 * z0 + z1 * z1
        logdet = jnp.log(l00) + jnp.log(l11)
        nll = 0.5 * maha + logdet + math.log(2.0 * math.pi)
        nll = jnp.clip(nll, -1e7, 1e7)
        acc_ref[b, 4:5, 0:1] += jnp.sum(nll, axis=0, keepdims=True)

        dd = pp - tp
        a = jnp.abs(dd)
        huber = jnp.where(a < HUBER_DELTA, 0.5 * dd * dd,
                          HUBER_DELTA * (a - 0.5 * HUBER_DELTA))
        acc_ref[b, 5:6, 0:1] += jnp.sum(huber, axis=(0, 1), keepdims=True)

    @pl.when(jnp.logical_and(b == B - 1, pt == NPT - 1))
    def _finalize():
        bce_sum = jnp.zeros((1, 1), jnp.float32)
        cls_sum = jnp.zeros((1, 1), jnp.float32)
        nll_sum = jnp.zeros((1, 1), jnp.float32)
        hub_sum = jnp.zeros((1, 1), jnp.float32)
        dice_sum = jnp.zeros((1, 1), jnp.float32)
        for bb in range(B):
            bce_sum += jnp.sum(acc_ref[bb, 0:1, 0:NE], axis=1, keepdims=True)
            num = jnp.sum(acc_ref[bb, 1:2, 0:NE], axis=1, keepdims=True)
            den = jnp.sum(acc_ref[bb, 2:3, 0:NE], axis=1, keepdims=True)
            dice_sum += 1.0 - (num + 1.0) / (den + 1.0)
            cls_sum += acc_ref[bb, 3:4, 0:1]
            nll_sum += acc_ref[bb, 4:5, 0:1]
            hub_sum += acc_ref[bb, 5:6, 0:1]
        total = (cls_sum / (B * Q)
                 + bce_sum / (B * P * NE)
                 + dice_sum / B
                 + nll_sum / (B * NE)
                 + hub_sum / (B * NE * 2))
        total_ref[...] = total


def kernel(pred_logits, pred_seg_logits, true_seg, pred_positions,
           pred_std_cholesky, true_positions, query_batch_offsets,
           electron_batch_offsets, matched_indices):
    logits3 = pred_logits.reshape(B, Q, 1)
    pos3 = pred_positions.reshape(B, Q, 2)
    chol3 = pred_std_cholesky.reshape(B, Q, 4)
    tpos3 = true_positions.reshape(B, NE, 2)

    grid = (B, NPT)
    acc, total = pl.pallas_call(
        _loss_kernel,
        grid=grid,
        in_specs=[
            pl.BlockSpec((1, 2, NE), lambda b, pt: (b, 0, 0)),
            pl.BlockSpec((1, Q, 1), lambda b, pt: (b, 0, 0)),
            pl.BlockSpec((1, Q, 2), lambda b, pt: (b, 0, 0)),
            pl.BlockSpec((1, Q, 4), lambda b, pt: (b, 0, 0)),
            pl.BlockSpec((1, NE, 2), lambda b, pt: (b, 0, 0)),
            pl.BlockSpec((1, TP, Q), lambda b, pt: (b, pt, 0)),
            pl.BlockSpec((1, TP, NE), lambda b, pt: (b, pt, 0)),
        ],
        out_specs=[
            pl.BlockSpec((B, 8, 128), lambda b, pt: (0, 0, 0)),
            pl.BlockSpec((1, 1), lambda b, pt: (0, 0)),
        ],
        out_shape=[
            jax.ShapeDtypeStruct((B, 8, 128), jnp.float32),
            jax.ShapeDtypeStruct((1, 1), jnp.float32),
        ],
    )(matched_indices, logits3, pos3, chol3, tpos3,
      pred_seg_logits, true_seg)
    return total[0, 0]
